# Initial kernel scaffold; baseline (speedup 1.0000x reference)
import jax
import jax.numpy as jnp
from jax.experimental import pallas as pl


def kernel(x_in, h_in, t, edge_indices, node_mask, edge_mask, params):
    raise NotImplementedError("write your pallas kernel here")


# speedup vs baseline: 9.9543x; 9.9543x over previous
import jax
import jax.numpy as jnp
from jax import lax
from jax.experimental import pallas as pl

N = 29
E = N * (N - 1)
DH = 5
HID = 256
NL = 4


def _egnn_kernel(x_ref, h_ref, t_ref, ei_ref, eit_ref, nm_ref, em_ref,
                 W_in_ref, b_in_ref, W_out_ref, b_out_ref,
                 We1_ref, be1_ref, We2_ref, be2_ref, Wa_ref, ba_ref,
                 Wh1_ref, bh1_ref, Wh2_ref, bh2_ref,
                 Wx1_ref, bx1_ref, Wx2_ref, bx2_ref, Wx3_ref,
                 out_ref):
    silu = jax.nn.silu
    f32 = jnp.float32
    x0 = x_ref[0]
    h_in = h_ref[0]
    tt = t_ref[0]
    nm = nm_ref[0]
    em = em_ref[0]
    idx = ei_ref[0]
    idxt = eit_ref[0]

    iota_en = lax.broadcasted_iota(jnp.int32, (E, N), 1)
    Pi = (idx[:, 0:1] == iota_en).astype(f32)
    Pj = (idx[:, 1:2] == iota_en).astype(f32)
    iota_ne = lax.broadcasted_iota(jnp.int32, (N, E), 0)
    PiT = (idxt[0:1, :] == iota_ne).astype(f32)

    def mm(a, b):
        return jnp.dot(a, b, preferred_element_type=f32)

    h = mm(h_in, W_in_ref[0:DH, :]) + tt * W_in_ref[DH:DH + 1, :] + b_in_ref[0:1, :]

    xi0 = mm(Pi, x0)
    xj0 = mm(Pj, x0)
    a0 = jnp.sqrt(jnp.sum((xi0 - xj0) ** 2, axis=1, keepdims=True)) * em

    x = x0
    for l in range(NL):
        xi = mm(Pi, x)
        xj = mm(Pj, x)
        diff = (xi - xj) * em
        d = jnp.sqrt(jnp.sum(diff * diff, axis=1, keepdims=True))
        dsq = d * d

        pre_x = (mm(Pi, mm(h, Wx1_ref[l, 0:HID, :]))
                 + mm(Pj, mm(h, Wx1_ref[l, HID:2 * HID, :]))
                 + dsq * Wx1_ref[l, 2 * HID:2 * HID + 1, :]
                 + a0 * Wx1_ref[l, 2 * HID + 1:2 * HID + 2, :]
                 + bx1_ref[l])
        m = silu(pre_x)
        m = silu(mm(m, Wx2_ref[l]) + bx2_ref[l])
        u = jnp.tanh(mm(m, Wx3_ref[l])) * 15.0
        ux = u * diff / (d + 1.0)
        x = (x + mm(PiT, ux)) * nm

        pre_e = (mm(Pi, mm(h, We1_ref[l, 0:HID, :]))
                 + mm(Pj, mm(h, We1_ref[l, HID:2 * HID, :]))
                 + dsq * We1_ref[l, 2 * HID:2 * HID + 1, :]
                 + a0 * We1_ref[l, 2 * HID + 1:2 * HID + 2, :]
                 + be1_ref[l])
        me = silu(pre_e)
        me = silu(mm(me, We2_ref[l]) + be2_ref[l])
        e = jax.nn.sigmoid(mm(me, Wa_ref[l]) + ba_ref[l])
        em_agg = mm(PiT, e * me)

        dh = mm(silu(mm(h, Wh1_ref[l, 0:HID, :])
                     + mm(em_agg, Wh1_ref[l, HID:2 * HID, :])
                     + bh1_ref[l]),
                Wh2_ref[l]) + bh2_ref[l]
        h = (h + dh) * nm

    x_out = (x - x0) * nm
    n_atoms = jnp.sum(nm, axis=0, keepdims=True)
    x_mean = jnp.sum(x_out, axis=0, keepdims=True) / n_atoms
    x_out = (x_out - x_mean) * nm
    h_out = (mm(h, W_out_ref[...]) + b_out_ref[0:1, :]) * nm
    out_ref[0] = jnp.concatenate([x_out, h_out[:, 0:DH]], axis=1)


def kernel(x_in, h_in, t, edge_indices, node_mask, edge_mask, params):
    B = x_in.shape[0]
    blocks = params["blocks"]

    def stack(name):
        return jnp.stack([blk[name] for blk in blocks])

    We1 = stack("We1")
    be1 = stack("be1").reshape(NL, 1, HID)
    We2 = stack("We2")
    be2 = stack("be2").reshape(NL, 1, HID)
    Wa = stack("Wa")
    ba = stack("ba").reshape(NL, 1, 1)
    Wh1 = stack("Wh1")
    bh1 = stack("bh1").reshape(NL, 1, HID)
    Wh2 = stack("Wh2")
    bh2 = stack("bh2").reshape(NL, 1, HID)
    Wx1 = stack("Wx1")
    bx1 = stack("bx1").reshape(NL, 1, HID)
    Wx2 = stack("Wx2")
    bx2 = stack("bx2").reshape(NL, 1, HID)
    Wx3 = stack("Wx3")

    W_in = params["W_in"]
    b_in = params["b_in"].reshape(1, HID)
    W_out = params["W_out"]
    b_out = params["b_out"].reshape(1, DH + 1)

    ei = edge_indices.astype(jnp.int32)
    eit = jnp.swapaxes(ei, 1, 2)

    def data_spec(shape):
        return pl.BlockSpec((1,) + shape, lambda b: (b, 0, 0))

    def const_spec(arr):
        nd = arr.ndim
        return pl.BlockSpec(arr.shape, lambda b, _n=nd: (0,) * _n)

    weights = [W_in, b_in, W_out, b_out,
               We1, be1, We2, be2, Wa, ba,
               Wh1, bh1, Wh2, bh2,
               Wx1, bx1, Wx2, bx2, Wx3]

    out = pl.pallas_call(
        _egnn_kernel,
        grid=(B,),
        in_specs=[data_spec((N, 3)), data_spec((N, DH)), data_spec((N, 1)),
                  data_spec((E, 2)), data_spec((2, E)),
                  data_spec((N, 1)), data_spec((E, 1))]
                 + [const_spec(w) for w in weights],
        out_specs=pl.BlockSpec((1, N, 3 + DH), lambda b: (b, 0, 0)),
        out_shape=jax.ShapeDtypeStruct((B, N, 3 + DH), jnp.float32),
    )(x_in, h_in, t, ei, eit, node_mask, edge_mask, *weights)
    return out
